# baseline (speedup 1.0000x reference)
import math

import jax
import jax.numpy as jnp
from jax.experimental import pallas as pl
from jax.experimental.pallas import tpu as pltpu

N = 100000
B = 64
F = 128
R = 16
T = 2000
SCALE = 1.0 / 0.6
INV3 = 1.0 / math.sqrt(3.0)
INVH = 1.0 / math.sqrt(float(F))


def _ssilu(v):
    return v * jax.nn.sigmoid(v) * SCALE


def _dot(a, b):
    return jax.lax.dot_general(a, b, (((1,), (0,)), ((), ())),
                               preferred_element_type=jnp.float32)


def _dott(a, b):
    return jax.lax.dot_general(a, b, (((0,), (0,)), ((), ())),
                               preferred_element_type=jnp.float32)


def _fused_kernel(x_ref, vec_ref, ef_ref, eu_ref, bt_ref, gl_ref, sel_ref,
                  wxp1_ref, bxp1_ref, wxp2_ref, bxp2_ref, wep_ref, bep_ref,
                  wsl1_ref, bsl1_ref, wsl2_ref, bsl2_ref, wvl_ref,
                  wsg1_ref, bsg1_ref, wsg2_ref, bsg2_ref, wvg_ref,
                  wvlp_ref, wslp1_ref, bslp1_ref, wslp2_ref, bslp2_ref,
                  wlt_ref,
                  hx_ref, hvec_ref, sl_ref, vl_ref, ld_ref,
                  seg_hx, seg_hv, cnt):
    i = pl.program_id(0)
    nsteps = pl.num_programs(0)

    @pl.when(i == 0)
    def _init():
        seg_hx[...] = jnp.zeros_like(seg_hx)
        seg_hv[...] = jnp.zeros_like(seg_hv)
        cnt[...] = jnp.zeros_like(cnt)

    xb = x_ref[...]
    h1 = _ssilu(_dot(xb, wxp1_ref[...]) + bxp1_ref[...])
    x_p = _dot(h1, wxp2_ref[...]) + bxp2_ref[...]
    edge_p = _dot(ef_ref[...], wep_ref[...]) + bep_ref[...]
    prod = x_p * edge_p * INV3
    x1 = prod[:, :F]
    x2 = prod[:, F:2 * F]
    xn = prod[:, 2 * F:] + xb

    eub = _dot(eu_ref[...], sel_ref[...])

    bt = bt_ref[...]
    onehot = (bt == jax.lax.broadcasted_iota(jnp.int32, (T, B), 1)
              ).astype(jnp.float32)
    gath = _dot(onehot, gl_ref[...])
    sl_g = gath[:, :F]

    hmid = _ssilu(_dot(xn, wsl1_ref[:F, :]) + _dot(sl_g, wsl1_ref[F:, :])
                  + bsl1_ref[...])
    hx = _ssilu(_dot(hmid, wsl2_ref[...]) + bsl2_ref[...]) + xn
    hx_ref[...] = hx
    seg_hx[...] += _dott(onehot, hx)

    for d in range(3):
        sl_d = slice(F * d, F * (d + 1))
        vecn_d = (x1 * vec_ref[:, d, :] + x2 * eub[:, sl_d]) * INVH
        vtmp = vecn_d + gath[:, F * (d + 1):F * (d + 2)]
        hv_d = _dot(vtmp, wvl_ref[...]) + vecn_d
        hvec_ref[:, d, :] = hv_d
        seg_hv[:, sl_d] += _dott(onehot, hv_d)

    cnt[...] += _dott(onehot, jnp.ones((T, 1), jnp.float32))

    @pl.when(i == nsteps - 1)
    def _global():
        rc = 1.0 / jnp.maximum(cnt[...], 1.0)
        gl = gl_ref[...]
        scalar_l = gl[:, :F]
        mean_x = seg_hx[...] * rc
        tg = _ssilu(_dot(mean_x, wsg1_ref[:F, :]) +
                    _dot(scalar_l, wsg1_ref[F:, :]) + bsg1_ref[...])
        slt = _ssilu(_dot(tg, wsg2_ref[...]) + bsg2_ref[...])
        sl0 = scalar_l + slt

        vl0 = []
        vh1 = []
        vnormsq = jnp.full((B, F), 1e-8, jnp.float32)
        for d in range(3):
            vl_d = gl[:, F * (d + 1):F * (d + 2)]
            mv_d = seg_hv[:, F * d:F * (d + 1)] * rc
            vl0_d = vl_d + _dot(mv_d + vl_d, wvg_ref[...])
            vh_d = _dot(vl0_d, wvlp_ref[...])
            vl0.append(vl0_d)
            vh1.append(vh_d[:, :F])
            vnormsq = vnormsq + vh_d[:, F:] * vh_d[:, F:]
        vnorm = jnp.sqrt(vnormsq)

        sh = _dot(_ssilu(_dot(sl0, wslp1_ref[:F, :]) +
                         _dot(vnorm, wslp1_ref[F:, :]) + bslp1_ref[...]),
                  wslp2_ref[...]) + bslp2_ref[...]
        sh1 = sh[:, :F]
        gate = jnp.tanh(sh[:, 2 * F:])
        sl_ref[...] = sh[:, F:2 * F] + sl0 * gate
        wlt = wlt_ref[...]
        for d in range(3):
            vlo_d = sh1 * vh1[d] + vl0[d]
            vl_ref[:, d, :] = vlo_d
            ld_ref[:, d:d + 1] = jnp.sum(vlo_d * wlt, axis=1, keepdims=True)


def kernel(x, scalar_l, vec, vector_l, edge_feat, edge_udiff, batch,
           Wxp1, bxp1, Wxp2, bxp2, Wep, bep, Wsl1, bsl1, Wsl2, bsl2, Wvl,
           Wsg1, bsg1, Wsg2, bsg2, Wvg, Wvlp, Wslp1, bslp1, Wslp2, bslp2, Wl):
    nsteps = N // T
    gl = jnp.concatenate([scalar_l, vector_l.reshape(B, 3 * F)], axis=1)
    bt = batch.reshape(N, 1)
    sel = jnp.repeat(jnp.eye(3, dtype=jnp.float32), F, axis=1)

    row = lambda i: (i, 0)
    rep = lambda i: (0, 0)

    in_specs = [
            pl.BlockSpec((T, F), row),
            pl.BlockSpec((T, 3, F), lambda i: (i, 0, 0)),
            pl.BlockSpec((T, R), row),
            pl.BlockSpec((T, 3), row),
            pl.BlockSpec((T, 1), row),
            pl.BlockSpec((B, 4 * F), rep),
            pl.BlockSpec((3, 3 * F), rep),
            pl.BlockSpec((F, F), rep),
            pl.BlockSpec((1, F), rep),
            pl.BlockSpec((F, 3 * F), rep),
            pl.BlockSpec((1, 3 * F), rep),
            pl.BlockSpec((R, 3 * F), rep),
            pl.BlockSpec((1, 3 * F), rep),
            pl.BlockSpec((2 * F, F), rep),
            pl.BlockSpec((1, F), rep),
            pl.BlockSpec((F, F), rep),
            pl.BlockSpec((1, F), rep),
            pl.BlockSpec((F, F), rep),
            pl.BlockSpec((2 * F, F), rep),
            pl.BlockSpec((1, F), rep),
            pl.BlockSpec((F, F), rep),
            pl.BlockSpec((1, F), rep),
            pl.BlockSpec((F, F), rep),
            pl.BlockSpec((F, 2 * F), rep),
            pl.BlockSpec((2 * F, F), rep),
            pl.BlockSpec((1, F), rep),
            pl.BlockSpec((F, 3 * F), rep),
            pl.BlockSpec((1, 3 * F), rep),
            pl.BlockSpec((1, F), rep),
    ]
    out_specs = [
            pl.BlockSpec((T, F), row),
            pl.BlockSpec((T, 3, F), lambda i: (i, 0, 0)),
            pl.BlockSpec((B, F), rep),
            pl.BlockSpec((B, 3, F), lambda i: (0, 0, 0)),
            pl.BlockSpec((B, 3), rep),
    ]

    hx, hvec, sl, vl, ld = pl.pallas_call(
        _fused_kernel,
        grid=(nsteps,),
        in_specs=in_specs,
        out_specs=out_specs,
        out_shape=[
            jax.ShapeDtypeStruct((N, F), jnp.float32),
            jax.ShapeDtypeStruct((N, 3, F), jnp.float32),
            jax.ShapeDtypeStruct((B, F), jnp.float32),
            jax.ShapeDtypeStruct((B, 3, F), jnp.float32),
            jax.ShapeDtypeStruct((B, 3), jnp.float32),
        ],
        scratch_shapes=[
            pltpu.VMEM((B, F), jnp.float32),
            pltpu.VMEM((B, 3 * F), jnp.float32),
            pltpu.VMEM((B, 1), jnp.float32),
        ],
        compiler_params=pltpu.CompilerParams(
            dimension_semantics=("arbitrary",),
        ),
    )(x, vec, edge_feat, edge_udiff, bt, gl, sel,
      Wxp1, bxp1.reshape(1, F), Wxp2, bxp2.reshape(1, 3 * F),
      Wep, bep.reshape(1, 3 * F), Wsl1, bsl1.reshape(1, F),
      Wsl2, bsl2.reshape(1, F), Wvl, Wsg1, bsg1.reshape(1, F),
      Wsg2, bsg2.reshape(1, F), Wvg, Wvlp, Wslp1, bslp1.reshape(1, F),
      Wslp2, bslp2.reshape(1, 3 * F), Wl.reshape(1, F))

    return (hx, hvec, sl, vl, ld.reshape(B, 3, 1))


# speedup vs baseline: 1.1312x; 1.1312x over previous
import math

import jax
import jax.numpy as jnp
from jax.experimental import pallas as pl
from jax.experimental.pallas import tpu as pltpu

N = 100000
B = 64
F = 128
R = 16
T = 2000
SCALE = 1.0 / 0.6
INV3 = 1.0 / math.sqrt(3.0)
INVH = 1.0 / math.sqrt(float(F))


def _ssilu(v):
    return v * jax.nn.sigmoid(v) * SCALE


def _dot(a, b):
    return jax.lax.dot_general(a, b, (((1,), (0,)), ((), ())),
                               preferred_element_type=jnp.float32)


def _dott(a, b):
    return jax.lax.dot_general(a, b, (((0,), (0,)), ((), ())),
                               preferred_element_type=jnp.float32)


def _vec_in_copy(vec_hbm, vbuf, vsem, step, slot, d):
    return pltpu.make_async_copy(
        vec_hbm.at[pl.ds(step * T, T), d, :], vbuf.at[slot, d],
        vsem.at[slot, d])


def _hv_out_copy(hvec_hbm, hbuf, hsem, step, slot, d):
    return pltpu.make_async_copy(
        hbuf.at[slot, d], hvec_hbm.at[pl.ds(step * T, T), d, :],
        hsem.at[slot, d])


def _fused_kernel(x_ref, vec_hbm, ef_ref, eu_ref, bt_ref, gl_ref, sel_ref,
                  wxp1_ref, bxp1_ref, wxp2_ref, bxp2_ref, wep_ref, bep_ref,
                  wsl1_ref, bsl1_ref, wsl2_ref, bsl2_ref, wvl_ref,
                  wsg1_ref, bsg1_ref, wsg2_ref, bsg2_ref, wvg_ref,
                  wvlp_ref, wslp1_ref, bslp1_ref, wslp2_ref, bslp2_ref,
                  wlt_ref,
                  hx_ref, hvec_hbm, sl_ref, vl_ref, ld_ref,
                  vbuf, hbuf, seg_hx, seg_hv, cnt, vsem, hsem):
    i = pl.program_id(0)
    nsteps = pl.num_programs(0)
    slot = jax.lax.rem(i, 2)
    nslot = 1 - slot

    @pl.when(i == 0)
    def _init():
        seg_hx[...] = jnp.zeros_like(seg_hx)
        seg_hv[...] = jnp.zeros_like(seg_hv)
        cnt[...] = jnp.zeros_like(cnt)
        for d in range(3):
            _vec_in_copy(vec_hbm, vbuf, vsem, 0, 0, d).start()

    @pl.when(i + 1 < nsteps)
    def _prefetch():
        for d in range(3):
            _vec_in_copy(vec_hbm, vbuf, vsem, i + 1, nslot, d).start()

    xb = x_ref[...]
    h1 = _ssilu(_dot(xb, wxp1_ref[...]) + bxp1_ref[...])
    x_p = _dot(h1, wxp2_ref[...]) + bxp2_ref[...]
    edge_p = _dot(ef_ref[...], wep_ref[...]) + bep_ref[...]
    prod = x_p * edge_p * INV3
    x1 = prod[:, :F]
    x2 = prod[:, F:2 * F]
    xn = prod[:, 2 * F:] + xb

    eub = _dot(eu_ref[...], sel_ref[...])

    bt = bt_ref[...]
    onehot = (bt == jax.lax.broadcasted_iota(jnp.int32, (T, B), 1)
              ).astype(jnp.float32)
    gath = _dot(onehot, gl_ref[...])
    sl_g = gath[:, :F]

    hmid = _ssilu(_dot(xn, wsl1_ref[:F, :]) + _dot(sl_g, wsl1_ref[F:, :])
                  + bsl1_ref[...])
    hx = _ssilu(_dot(hmid, wsl2_ref[...]) + bsl2_ref[...]) + xn
    hx_ref[...] = hx
    seg_hx[...] += _dott(onehot, hx)
    cnt[...] += _dott(onehot, jnp.ones((T, 1), jnp.float32))

    @pl.when(i >= 2)
    def _reclaim():
        for d in range(3):
            _hv_out_copy(hvec_hbm, hbuf, hsem, i - 2, slot, d).wait()

    for d in range(3):
        _vec_in_copy(vec_hbm, vbuf, vsem, i, slot, d).wait()
        sl_d = slice(F * d, F * (d + 1))
        vecn_d = (x1 * vbuf[slot, d] + x2 * eub[:, sl_d]) * INVH
        vtmp = vecn_d + gath[:, F * (d + 1):F * (d + 2)]
        hv_d = _dot(vtmp, wvl_ref[...]) + vecn_d
        hbuf[slot, d] = hv_d
        seg_hv[:, sl_d] += _dott(onehot, hv_d)
    for d in range(3):
        _hv_out_copy(hvec_hbm, hbuf, hsem, i, slot, d).start()

    @pl.when(i == nsteps - 1)
    def _drain():
        for d in range(3):
            _hv_out_copy(hvec_hbm, hbuf, hsem, i - 1, nslot, d).wait()
            _hv_out_copy(hvec_hbm, hbuf, hsem, i, slot, d).wait()

    @pl.when(i == nsteps - 1)
    def _global():
        rc = 1.0 / jnp.maximum(cnt[...], 1.0)
        gl = gl_ref[...]
        scalar_l = gl[:, :F]
        mean_x = seg_hx[...] * rc
        tg = _ssilu(_dot(mean_x, wsg1_ref[:F, :]) +
                    _dot(scalar_l, wsg1_ref[F:, :]) + bsg1_ref[...])
        slt = _ssilu(_dot(tg, wsg2_ref[...]) + bsg2_ref[...])
        sl0 = scalar_l + slt

        vl0 = []
        vh1 = []
        vnormsq = jnp.full((B, F), 1e-8, jnp.float32)
        for k in range(3):
            vl_k = gl[:, F * (k + 1):F * (k + 2)]
            mv_k = seg_hv[:, F * k:F * (k + 1)] * rc
            vl0_k = vl_k + _dot(mv_k + vl_k, wvg_ref[...])
            vh_k = _dot(vl0_k, wvlp_ref[...])
            vl0.append(vl0_k)
            vh1.append(vh_k[:, :F])
            vnormsq = vnormsq + vh_k[:, F:] * vh_k[:, F:]
        vnorm = jnp.sqrt(vnormsq)

        sh = _dot(_ssilu(_dot(sl0, wslp1_ref[:F, :]) +
                         _dot(vnorm, wslp1_ref[F:, :]) + bslp1_ref[...]),
                  wslp2_ref[...]) + bslp2_ref[...]
        sh1 = sh[:, :F]
        gate = jnp.tanh(sh[:, 2 * F:])
        sl_ref[...] = sh[:, F:2 * F] + sl0 * gate
        wlt = wlt_ref[...]
        for k in range(3):
            vlo_k = sh1 * vh1[k] + vl0[k]
            vl_ref[:, k, :] = vlo_k
            ld_ref[:, k:k + 1] = jnp.sum(vlo_k * wlt, axis=1, keepdims=True)


def kernel(x, scalar_l, vec, vector_l, edge_feat, edge_udiff, batch,
           Wxp1, bxp1, Wxp2, bxp2, Wep, bep, Wsl1, bsl1, Wsl2, bsl2, Wvl,
           Wsg1, bsg1, Wsg2, bsg2, Wvg, Wvlp, Wslp1, bslp1, Wslp2, bslp2, Wl):
    nsteps = N // T
    gl = jnp.concatenate([scalar_l, vector_l.reshape(B, 3 * F)], axis=1)
    bt = batch.reshape(N, 1)
    sel = jnp.repeat(jnp.eye(3, dtype=jnp.float32), F, axis=1)

    row = lambda i: (i, 0)
    rep = lambda i: (0, 0)

    in_specs = [
            pl.BlockSpec((T, F), row),
            pl.BlockSpec(memory_space=pl.ANY),
            pl.BlockSpec((T, R), row),
            pl.BlockSpec((T, 3), row),
            pl.BlockSpec((T, 1), row),
            pl.BlockSpec((B, 4 * F), rep),
            pl.BlockSpec((3, 3 * F), rep),
            pl.BlockSpec((F, F), rep),
            pl.BlockSpec((1, F), rep),
            pl.BlockSpec((F, 3 * F), rep),
            pl.BlockSpec((1, 3 * F), rep),
            pl.BlockSpec((R, 3 * F), rep),
            pl.BlockSpec((1, 3 * F), rep),
            pl.BlockSpec((2 * F, F), rep),
            pl.BlockSpec((1, F), rep),
            pl.BlockSpec((F, F), rep),
            pl.BlockSpec((1, F), rep),
            pl.BlockSpec((F, F), rep),
            pl.BlockSpec((2 * F, F), rep),
            pl.BlockSpec((1, F), rep),
            pl.BlockSpec((F, F), rep),
            pl.BlockSpec((1, F), rep),
            pl.BlockSpec((F, F), rep),
            pl.BlockSpec((F, 2 * F), rep),
            pl.BlockSpec((2 * F, F), rep),
            pl.BlockSpec((1, F), rep),
            pl.BlockSpec((F, 3 * F), rep),
            pl.BlockSpec((1, 3 * F), rep),
            pl.BlockSpec((1, F), rep),
    ]
    out_specs = [
            pl.BlockSpec((T, F), row),
            pl.BlockSpec(memory_space=pl.ANY),
            pl.BlockSpec((B, F), rep),
            pl.BlockSpec((B, 3, F), lambda i: (0, 0, 0)),
            pl.BlockSpec((B, 3), rep),
    ]

    hx, hvec, sl, vl, ld = pl.pallas_call(
        _fused_kernel,
        grid=(nsteps,),
        in_specs=in_specs,
        out_specs=out_specs,
        out_shape=[
            jax.ShapeDtypeStruct((N, F), jnp.float32),
            jax.ShapeDtypeStruct((N, 3, F), jnp.float32),
            jax.ShapeDtypeStruct((B, F), jnp.float32),
            jax.ShapeDtypeStruct((B, 3, F), jnp.float32),
            jax.ShapeDtypeStruct((B, 3), jnp.float32),
        ],
        scratch_shapes=[
            pltpu.VMEM((2, 3, T, F), jnp.float32),
            pltpu.VMEM((2, 3, T, F), jnp.float32),
            pltpu.VMEM((B, F), jnp.float32),
            pltpu.VMEM((B, 3 * F), jnp.float32),
            pltpu.VMEM((B, 1), jnp.float32),
            pltpu.SemaphoreType.DMA((2, 3)),
            pltpu.SemaphoreType.DMA((2, 3)),
        ],
        compiler_params=pltpu.CompilerParams(
            dimension_semantics=("arbitrary",),
        ),
    )(x, vec, edge_feat, edge_udiff, bt, gl, sel,
      Wxp1, bxp1.reshape(1, F), Wxp2, bxp2.reshape(1, 3 * F),
      Wep, bep.reshape(1, 3 * F), Wsl1, bsl1.reshape(1, F),
      Wsl2, bsl2.reshape(1, F), Wvl, Wsg1, bsg1.reshape(1, F),
      Wsg2, bsg2.reshape(1, F), Wvg, Wvlp, Wslp1, bslp1.reshape(1, F),
      Wslp2, bslp2.reshape(1, 3 * F), Wl.reshape(1, F))

    return (hx, hvec, sl, vl, ld.reshape(B, 3, 1))
